# baseline (speedup 1.0000x reference)
import jax
import jax.numpy as jnp
from jax import lax
from jax.experimental import pallas as pl
from jax.experimental.pallas import tpu as pltpu
from jax.experimental.pallas import tpu_sc as plsc

NC = 2
NS = 16
L = 16


def _mm_body(x_ref, w_ref, o_ref):
    o_ref[...] = jnp.dot(x_ref[...], w_ref[...],
                         preferred_element_type=jnp.float32)


def _matmul(x, W):
    M, K = x.shape
    _, Nf = W.shape
    BLK = 2000
    assert M % BLK == 0
    return pl.pallas_call(
        _mm_body,
        grid=(M // BLK,),
        in_specs=[pl.BlockSpec((BLK, K), lambda i: (i, 0)),
                  pl.BlockSpec((K, Nf), lambda i: (0, 0))],
        out_specs=pl.BlockSpec((BLK, Nf), lambda i: (i, 0)),
        out_shape=jax.ShapeDtypeStruct((M, Nf), jnp.float32),
    )(x, W)


def _make_edge_kernel(B, N, E, F):
    N2 = N // NC
    EPT = E // NS
    CH = 2000
    assert EPT % CH == 0
    NV = CH // L
    GR = 16
    NBUF = 8
    LSZ = CH + NBUF * GR
    CBITS = 15
    assert B * N <= (1 << CBITS) and N2 * (1 << CBITS) < 2 ** 31
    CMASK = (1 << CBITS) - 1
    ACC_ROWS = 5024
    assert ACC_ROWS >= N2 + 1 and ACC_ROWS % 16 == 0
    NB16 = ACC_ROWS // 16
    NB8 = N2 // 8
    assert N2 % 8 == 0
    FL = F // L

    mesh = plsc.VectorSubcoreMesh(core_axis_name="c", subcore_axis_name="s",
                                  num_cores=NC, num_subcores=NS)

    def body(support, eids, ew, bias_hbm, out,
             rows_v, cols_v, w_v, code_l, w_l,
             gbufs, gidxs, sidxs, brep, accum, gsems, ssems, sem):
        c = lax.axis_index("c")
        s = lax.axis_index("s")
        base = c * N2

        for r in range(16):
            pltpu.sync_copy(bias_hbm, brep.at[r])

        def batch_body(b, carry0):
            def init_body(j, carry):
                blk = s + j * NS
                @pl.when(blk < NB16)
                def _():
                    pltpu.sync_copy(brep, accum.at[pl.ds(blk * 16, 16)])
                return carry
            lax.fori_loop(0, NB16 // NS + 1, init_body, jnp.int32(0))
            plsc.subcore_barrier()

            def stage_body(k, carry):
                off = s * EPT + k * CH
                pltpu.sync_copy(eids.at[b, 0, pl.ds(off, CH)], rows_v)
                pltpu.sync_copy(eids.at[b, 1, pl.ds(off, CH)], cols_v)
                pltpu.sync_copy(ew.at[b, pl.ds(off, CH)], w_v)

                def scan_body(i, cnt):
                    rv = rows_v[pl.ds(i * L, L)]
                    cv = cols_v[pl.ds(i * L, L)]
                    wv = w_v[pl.ds(i * L, L)]
                    u = rv - base
                    m = (u >= 0) & (u < N2)
                    mi = m.astype(jnp.int32)
                    pos = cnt + plsc.cumsum(mi) - 1
                    code = (u << CBITS) | (cv + b * N)
                    plsc.store_scatter(code_l, [pos], code, mask=m)
                    plsc.store_scatter(w_l, [pos], wv, mask=m)
                    return cnt + jnp.sum(mi)

                cnt = lax.fori_loop(0, NV, scan_body, jnp.int32(0))

                dummy = jnp.full((L,), N2 << CBITS, jnp.int32)
                for q in range(GR // L):
                    code_l[pl.ds(cnt + q * L, L)] = dummy
                    w_l[pl.ds(cnt + q * L, L)] = jnp.zeros((L,), jnp.float32)

                n_ch = (cnt + GR - 1) // GR
                n_quad = (n_ch + NBUF - 1) // NBUF

                def set_gidx(gx, j):
                    for q in range(GR // L):
                        code = code_l[pl.ds(j * GR + q * L, L)]
                        gx[pl.ds(q * L, L)] = code & CMASK

                def set_sidx(sx, j):
                    for q in range(GR // L):
                        code = code_l[pl.ds(j * GR + q * L, L)]
                        sx[pl.ds(q * L, L)] = \
                            lax.shift_right_logical(code, CBITS)

                def scale_ring(i, j):
                    def row_body(r, carry2):
                        wbc = plsc.load_gather(
                            w_l, [jnp.full((L,), j * GR + r, jnp.int32)])
                        for f in range(FL):
                            gbufs[i, r, pl.ds(f * L, L)] = \
                                gbufs[i, r, pl.ds(f * L, L)] * wbc
                        return carry2
                    lax.fori_loop(0, GR, row_body, jnp.int32(0))

                for i in range(NBUF - 1):
                    @pl.when(i < n_ch)
                    def _():
                        set_gidx(gidxs.at[i], i)
                        pltpu.async_copy(support.at[gidxs.at[i]],
                                         gbufs.at[i], gsems.at[i])

                def quad_body(p, carry):
                    a = NBUF * p
                    for i in range(NBUF):
                        j = a + i
                        gb = gbufs.at[i]

                        @pl.when(j < n_ch)
                        def _():
                            pltpu.make_async_copy(support.at[gidxs.at[i]],
                                                  gb, gsems.at[i]).wait()
                            scale_ring(i, j)
                            set_sidx(sidxs.at[i], j)
                            pltpu.async_copy(gb, accum.at[sidxs.at[i]],
                                             ssems.at[i], add=True)
                        i1 = (i - 1) % NBUF
                        @pl.when((j >= 1) & (j <= n_ch))
                        def _():
                            pltpu.make_async_copy(
                                gbufs.at[i1], accum.at[sidxs.at[i1]],
                                ssems.at[i1]).wait()
                        i3 = (i + NBUF - 1) % NBUF
                        @pl.when(j + NBUF - 1 < n_ch)
                        def _():
                            set_gidx(gidxs.at[i3], j + NBUF - 1)
                            pltpu.async_copy(support.at[gidxs.at[i3]],
                                             gbufs.at[i3], gsems.at[i3])
                    return carry

                lax.fori_loop(0, n_quad, quad_body, jnp.int32(0))
                @pl.when((n_ch > 0) & (n_ch % NBUF == 0))
                def _():
                    i_last = NBUF - 1
                    pltpu.make_async_copy(gbufs.at[i_last],
                                          accum.at[sidxs.at[i_last]],
                                          ssems.at[i_last]).wait()
                return carry

            lax.fori_loop(0, EPT // CH, stage_body, jnp.int32(0))
            plsc.subcore_barrier()

            out_base = b * N + c * N2

            def wout_body(j, carry):
                blk = s + j * NS
                @pl.when(blk < NB8)
                def _():
                    pltpu.sync_copy(
                        accum.at[pl.ds(blk * 8, 8)],
                        out.at[pl.ds(out_base + blk * 8, 8)])
                return carry
            lax.fori_loop(0, NB8 // NS + 1, wout_body, jnp.int32(0))
            plsc.subcore_barrier()
            return carry0

        lax.fori_loop(0, B, batch_body, jnp.int32(0))

    return pl.kernel(
        body,
        out_type=jax.ShapeDtypeStruct((B * N, F), jnp.float32),
        mesh=mesh,
        compiler_params=pltpu.CompilerParams(use_tc_tiling_on_sc=False,
                                             needs_layout_passes=False),
        scratch_types=[
            pltpu.VMEM((CH,), jnp.int32),
            pltpu.VMEM((CH,), jnp.int32),
            pltpu.VMEM((CH,), jnp.float32),
            pltpu.VMEM((LSZ,), jnp.int32),
            pltpu.VMEM((LSZ,), jnp.float32),
            pltpu.VMEM((NBUF, GR, F), jnp.float32),
            pltpu.VMEM((NBUF, GR), jnp.int32),
            pltpu.VMEM((NBUF, GR), jnp.int32),
            pltpu.VMEM((16, F), jnp.float32),
            pltpu.VMEM_SHARED((ACC_ROWS, F), jnp.float32),
            pltpu.SemaphoreType.DMA((NBUF,)),
            pltpu.SemaphoreType.DMA((NBUF,)),
            pltpu.SemaphoreType.DMA,
        ],
    )


def kernel(input, edge_ids, edge_weights, W, bias):
    B, N, IN_F = input.shape
    OUT_F = W.shape[1]
    E = edge_weights.shape[1]
    support = _matmul(input.reshape(B * N, IN_F), W)
    edge_k = _make_edge_kernel(B, N, E, OUT_F)
    out = edge_k(support, edge_ids, edge_weights, bias)
    return out.reshape(B, N, OUT_F)
